# Initial kernel scaffold; baseline (speedup 1.0000x reference)
import jax
import jax.numpy as jnp
from jax.experimental import pallas as pl


def kernel(query_coords, sensor_coords, sensor_features):
    raise NotImplementedError("write your pallas kernel here")


# speedup vs baseline: 20.7235x; 20.7235x over previous
import jax
import jax.numpy as jnp
from jax import lax
from jax.experimental import pallas as pl

K = 8
EPS = 1e-8
N_S = 2048
N_F = 256
QT = 256


def _tc_body(q_ref, sT_ref, f_ref, o_ref):
    qq = q_ref[0]
    sT = sT_ref[...]
    qs = lax.dot_general(
        qq.astype(jnp.bfloat16),
        sT.astype(jnp.bfloat16),
        (((1,), (0,)), ((), ())),
        preferred_element_type=jnp.float32,
    )
    q2 = jnp.sum(qq * qq, axis=1, keepdims=True)
    s2 = jnp.sum(sT * sT, axis=0, keepdims=True)
    d2 = (q2 + s2) - 2.0 * qs
    d2 = jnp.maximum(d2, 0.0)
    iota = lax.broadcasted_iota(jnp.int32, (QT, N_S), 1)
    w_mat = jnp.zeros((QT, N_S), jnp.float32)
    w_sum = jnp.zeros((QT, 1), jnp.float32)
    for _k in range(K):
        m = jnp.min(d2, axis=1, keepdims=True)
        cand = jnp.where(d2 == m, iota, N_S)
        idx = jnp.min(cand, axis=1, keepdims=True)
        onehot = iota == idx
        w = 1.0 / (jnp.sqrt(m + 1e-12) + EPS)
        w_mat = jnp.where(onehot, w, w_mat)
        w_sum = w_sum + w
        d2 = jnp.where(onehot, jnp.inf, d2)
    w_mat = w_mat / w_sum
    o_ref[0] = jnp.dot(w_mat, f_ref[0], preferred_element_type=jnp.float32)


@jax.jit
def kernel(query_coords, sensor_coords, sensor_features):
    B, n_q, _ = query_coords.shape
    s_t = sensor_coords.T
    return pl.pallas_call(
        _tc_body,
        grid=(B, n_q // QT),
        in_specs=[
            pl.BlockSpec((1, QT, 3), lambda b, t: (b, t, 0)),
            pl.BlockSpec((3, N_S), lambda b, t: (0, 0)),
            pl.BlockSpec((1, N_S, N_F), lambda b, t: (b, 0, 0)),
        ],
        out_specs=pl.BlockSpec((1, QT, N_F), lambda b, t: (b, t, 0)),
        out_shape=jax.ShapeDtypeStruct((B, n_q, N_F), jnp.float32),
    )(query_coords, s_t, sensor_features)
